# Initial kernel scaffold; baseline (speedup 1.0000x reference)
import jax
import jax.numpy as jnp
from jax.experimental import pallas as pl


def kernel(inputs, W):
    raise NotImplementedError("write your pallas kernel here")


# speedup vs baseline: 1.7606x; 1.7606x over previous
import functools

import jax
import jax.numpy as jnp
from jax import lax
from jax.experimental import pallas as pl
from jax.experimental.pallas import tpu as pltpu
from jax.experimental.pallas import tpu_sc as plsc

_TOP_K = 10
_LABEL = 7
_C = 1000
_B = 128
_D = 12288
_KBLK = 1536
_NKB = _D // _KBLK
_NW = 32
_SPW = _B // _NW
_ROWS = _TOP_K
_NEG = -3.0e38


def _mm_topk_body(x_ref, w_ref, f_ref, idx_ref, acc_ref):
    j = pl.program_id(0)

    @pl.when(j == 0)
    def _():
        acc_ref[...] = jnp.zeros_like(acc_ref)

    acc_ref[...] += lax.dot_general(
        x_ref[...], w_ref[...],
        (((1,), (1,)), ((), ())),
        preferred_element_type=jnp.float32)

    @pl.when(j == _NKB - 1)
    def _():
        logits = acc_ref[...]
        tg = logits[:, _LABEL:_LABEL + 1]
        iota = lax.broadcasted_iota(jnp.int32, (_B, _C), 1)
        work = logits
        vals, idxs = [], []
        for _ in range(_TOP_K):
            m = jnp.max(work, axis=1, keepdims=True)
            sel = jnp.where(work == m, iota, _C)
            ix = jnp.min(sel, axis=1, keepdims=True)
            vals.append(m)
            idxs.append(ix)
            work = jnp.where(iota == ix, _NEG, work)
        f = tg - sum(vals[1:]) * (1.0 / 9.0)
        f_ref[...] = jnp.broadcast_to(f, (_B, 128))
        mat = jnp.concatenate(
            [jnp.full((_B, 1), _LABEL, jnp.int32)] + idxs[1:], axis=1)
        pad = jnp.zeros((_B, 64 - _ROWS), jnp.int32)
        idx_ref[...] = jnp.concatenate([mat, pad], axis=1)


def _mm_topk(x2, W):
    return pl.pallas_call(
        _mm_topk_body,
        grid=(_NKB,),
        in_specs=[
            pl.BlockSpec((_B, _KBLK), lambda j: (0, j)),
            pl.BlockSpec((_C, _KBLK), lambda j: (0, j)),
        ],
        out_specs=[
            pl.BlockSpec((_B, 128), lambda j: (0, 0)),
            pl.BlockSpec((_B, 64), lambda j: (0, 0)),
        ],
        out_shape=[
            jax.ShapeDtypeStruct((_B, 128), jnp.float32),
            jax.ShapeDtypeStruct((_B, 64), jnp.int32),
        ],
        scratch_shapes=[pltpu.VMEM((_B, _C), jnp.float32)],
    )(x2, W)


def _sc_norm_body(w_hbm, idx_hbm, out_hbm, idx_v, bufA, bufB, out_v, semA, semB):
    cid = lax.axis_index("c")
    sid = lax.axis_index("s")
    wid = sid * 2 + cid
    pltpu.sync_copy(idx_hbm.at[pl.ds(wid * _SPW, _SPW)], idx_v)

    for k in range(_SPW):
        cA = pltpu.async_copy(w_hbm.at[idx_v.at[k, pl.ds(0, 8)]], bufA, semA)
        cB = pltpu.async_copy(w_hbm.at[idx_v.at[k, pl.ds(8, 2)]], bufB, semB)
        cA.wait()
        cB.wait()

        def body(i, a):
            off = pl.multiple_of(i * 16, 16)
            w7 = bufA[0, pl.ds(off, 16)]
            u = bufA[1, pl.ds(off, 16)]
            for r in range(2, 8):
                u = u + bufA[r, pl.ds(off, 16)]
            u = u + bufB[0, pl.ds(off, 16)] + bufB[1, pl.ds(off, 16)]
            d = w7 - u * (1.0 / 9.0)
            return a + d * d

        acc = lax.fori_loop(0, _D // 16, body, jnp.zeros((16,), jnp.float32))
        out_v[k] = acc
    pltpu.sync_copy(out_v, out_hbm.at[pl.ds(wid * _SPW, _SPW)])


def _sc_norm(W, idx):
    mesh = plsc.VectorSubcoreMesh(
        core_axis_name="c", subcore_axis_name="s",
        num_cores=2, num_subcores=16)
    fn = functools.partial(
        pl.kernel, mesh=mesh,
        out_type=jax.ShapeDtypeStruct((_B, 16), jnp.float32),
        scratch_types=[
            pltpu.VMEM((_SPW, _ROWS), jnp.int32),
            pltpu.VMEM((8, _D), jnp.float32),
            pltpu.VMEM((2, _D), jnp.float32),
            pltpu.VMEM((_SPW, 16), jnp.float32),
            pltpu.SemaphoreType.DMA,
            pltpu.SemaphoreType.DMA,
        ],
    )(_sc_norm_body)
    return fn(W, idx)


def kernel(inputs, W):
    x2 = inputs.reshape(_B, _D)
    f_out, idx_out = _mm_topk(x2, W)
    f = f_out[:, 0]
    idx = idx_out[:, :_ROWS]
    nrm = _sc_norm(W, idx)
    norm2 = jnp.sum(nrm, axis=1)
    return jnp.sum(f / jnp.sqrt(norm2))


# speedup vs baseline: 2.1972x; 1.2480x over previous
import functools

import jax
import jax.numpy as jnp
from jax import lax
from jax.experimental import pallas as pl
from jax.experimental.pallas import tpu as pltpu
from jax.experimental.pallas import tpu_sc as plsc

_TOP_K = 10
_LABEL = 7
_C = 1000
_CP = 1024
_B = 128
_D = 12288
_KBLK = 1536
_NKB = _D // _KBLK
_NW = 32
_SPW = _B // _NW
_ROWS = _TOP_K
_NEG = -3.0e38


def _mm_topk_body(x_ref, w_ref, f_ref, idx_ref, g_ref, y_ref, acc_ref):
    j = pl.program_id(0)

    @pl.when(j == 0)
    def _():
        acc_ref[...] = jnp.zeros_like(acc_ref)
        g_ref[...] = jnp.zeros_like(g_ref)

    w = w_ref[...]
    acc_ref[...] += lax.dot_general(
        x_ref[...], w,
        (((1,), (1,)), ((), ())),
        preferred_element_type=jnp.float32)

    wb = w.astype(jnp.bfloat16)
    g_ref[:, :_C] += lax.dot_general(
        wb, wb,
        (((1,), (1,)), ((), ())),
        preferred_element_type=jnp.float32)

    @pl.when(j == _NKB - 1)
    def _():
        logits = acc_ref[...]
        tg = logits[:, _LABEL:_LABEL + 1]
        iota = lax.broadcasted_iota(jnp.int32, (_B, _C), 1)
        work = logits
        vals, idxs = [], []
        for _ in range(_TOP_K):
            m = jnp.max(work, axis=1, keepdims=True)
            sel = jnp.where(work == m, iota, _C)
            ix = jnp.min(sel, axis=1, keepdims=True)
            vals.append(m)
            idxs.append(ix)
            work = jnp.where(iota == ix, _NEG, work)
        f = tg - sum(vals[1:]) * (1.0 / 9.0)
        f_ref[...] = jnp.broadcast_to(f, (_B, 128))
        mat = jnp.concatenate(
            [jnp.full((_B, 1), _LABEL, jnp.int32)] + idxs[1:], axis=1)
        pad = jnp.zeros((_B, 64 - _ROWS), jnp.int32)
        idx_ref[...] = jnp.concatenate([mat, pad], axis=1)
        iota2 = lax.broadcasted_iota(jnp.int32, (_B, _CP), 1)
        y = (iota2 == _LABEL).astype(jnp.float32)
        for ix in idxs[1:]:
            y = y - (1.0 / 9.0) * (iota2 == ix).astype(jnp.float32)
        y_ref[...] = y


def _mm_topk(x2, W):
    return pl.pallas_call(
        _mm_topk_body,
        grid=(_NKB,),
        in_specs=[
            pl.BlockSpec((_B, _KBLK), lambda j: (0, j)),
            pl.BlockSpec((_C, _KBLK), lambda j: (0, j)),
        ],
        out_specs=[
            pl.BlockSpec((_B, 128), lambda j: (0, 0)),
            pl.BlockSpec((_B, 64), lambda j: (0, 0)),
            pl.BlockSpec((_C, _CP), lambda j: (0, 0)),
            pl.BlockSpec((_B, _CP), lambda j: (0, 0)),
        ],
        out_shape=[
            jax.ShapeDtypeStruct((_B, 128), jnp.float32),
            jax.ShapeDtypeStruct((_B, 64), jnp.int32),
            jax.ShapeDtypeStruct((_C, _CP), jnp.float32),
            jax.ShapeDtypeStruct((_B, _CP), jnp.float32),
        ],
        scratch_shapes=[pltpu.VMEM((_B, _C), jnp.float32)],
    )(x2, W)


def _sc_norm_body(g_hbm, idx_hbm, y_hbm, out_hbm,
                  idx_v, y_v, bufA, bufB, out_v, semA, semB):
    cid = lax.axis_index("c")
    sid = lax.axis_index("s")
    wid = sid * 2 + cid
    pltpu.sync_copy(idx_hbm.at[pl.ds(wid * _SPW, _SPW)], idx_v)
    pltpu.sync_copy(y_hbm.at[pl.ds(wid * _SPW, _SPW)], y_v)

    for k in range(_SPW):
        cA = pltpu.async_copy(g_hbm.at[idx_v.at[k, pl.ds(0, 8)]], bufA, semA)
        cB = pltpu.async_copy(g_hbm.at[idx_v.at[k, pl.ds(8, 2)]], bufB, semB)
        cA.wait()
        cB.wait()

        def body(i, acc):
            off = pl.multiple_of(i * 16, 16)
            u = bufA[1, pl.ds(off, 16)]
            for r in range(2, 8):
                u = u + bufA[r, pl.ds(off, 16)]
            u = u + bufB[0, pl.ds(off, 16)] + bufB[1, pl.ds(off, 16)]
            t = bufA[0, pl.ds(off, 16)] - u * (1.0 / 9.0)
            return acc + t * y_v[k, pl.ds(off, 16)]

        acc = lax.fori_loop(0, _CP // 16, body, jnp.zeros((16,), jnp.float32))
        out_v[k] = acc
    pltpu.sync_copy(out_v, out_hbm.at[pl.ds(wid * _SPW, _SPW)])


def _sc_norm(G, idx, Y):
    mesh = plsc.VectorSubcoreMesh(
        core_axis_name="c", subcore_axis_name="s",
        num_cores=2, num_subcores=16)
    fn = functools.partial(
        pl.kernel, mesh=mesh,
        out_type=jax.ShapeDtypeStruct((_B, 16), jnp.float32),
        scratch_types=[
            pltpu.VMEM((_SPW, 16), jnp.int32),
            pltpu.VMEM((_SPW, _CP), jnp.float32),
            pltpu.VMEM((8, _CP), jnp.float32),
            pltpu.VMEM((2, _CP), jnp.float32),
            pltpu.VMEM((_SPW, 16), jnp.float32),
            pltpu.SemaphoreType.DMA,
            pltpu.SemaphoreType.DMA,
        ],
    )(_sc_norm_body)
    return fn(G, idx, Y)


def kernel(inputs, W):
    x2 = inputs.reshape(_B, _D)
    f_out, idx_out, G, Y = _mm_topk(x2, W)
    f = f_out[:, 0]
    idx = idx_out[:, :16]
    nrm = _sc_norm(G, idx, Y)
    norm2 = jnp.sum(nrm, axis=1)
    return jnp.sum(f / jnp.sqrt(norm2))
